# baseline (speedup 1.0000x reference)
import functools

import jax
import jax.numpy as jnp
from jax import lax
from jax.experimental import pallas as pl
from jax.experimental.pallas import tpu as pltpu
from jax.experimental.pallas import tpu_sc as plsc

POP = 16384
GL = 64
KEEP = 4096
CONT = 40
CHILDREN = POP - KEEP

_I32_MIN = -2147483648
_I32_MAX = 2147483647


def _iota16():
  return lax.iota(jnp.int32, 16)


def _splat(x):
  return jnp.zeros((16,), jnp.int32) + x


def _to_us(x_f32):
  b = plsc.bitcast(x_f32, jnp.int32)
  m = lax.shift_right_arithmetic(b, 31)
  return b ^ (m | _I32_MIN)


def _scan_bias():
  zc, _ = plsc.scan_count(jnp.zeros((16,), jnp.int32))
  return jnp.max(zc) - 15


def _sort_body(gene_hbm, tgt_hbm, kept_out, ssd_out, fit_out,
               ssd_sh, ord_sh, grows_v, tgt_v, ssd_tile_v,
               keys_a, vals_a, keys_b, vals_b,
               hist_v, cnt_v, fit_v, idx_v, sem):
  wid = lax.axis_index("s")
  bias = _scan_bias()

  pltpu.sync_copy(tgt_hbm, tgt_v)
  base_row = wid * 1024
  for chunk in range(4):
    pltpu.sync_copy(gene_hbm.at[pl.ds(base_row + chunk * 256, 256)], grows_v)

    def grp_body(g16, _, chunk=chunk):
      row_ids = _iota16() + g16 * 16

      def gene_body(g, acc):
        col = plsc.load_gather(grows_v, [row_ids, _splat(g)])
        t = plsc.load_gather(tgt_v, [_splat(g)])
        d = col - t
        return acc + d * d

      acc = lax.fori_loop(0, GL, gene_body, jnp.zeros((16,), jnp.float32))
      ssd_tile_v[pl.ds(chunk * 256 + g16 * 16, 16)] = acc.astype(jnp.int32)
      return 0

    lax.fori_loop(0, 16, grp_body, 0)
  pltpu.sync_copy(ssd_tile_v, ssd_sh.at[pl.ds(base_row, 1024)])
  plsc.subcore_barrier()

  @pl.when(wid == 0)
  def _():
    pltpu.sync_copy(ssd_sh, keys_a)

    @plsc.parallel_loop(0, 1024, 1, unroll=4)
    def _init_vals(i):
      vals_a[pl.ds(i * 16, 16)] = _iota16() + i * 16

    bufs = [(keys_a, vals_a, keys_b, vals_b),
            (keys_b, vals_b, keys_a, vals_a)]
    for p, (sk, sv, dk, dv) in enumerate(bufs):
      shift = p * 11

      def clr(i, _):
        hist_v[pl.ds(i * 16, 16)] = jnp.zeros((16,), jnp.int32)
        return 0

      lax.fori_loop(0, 128, clr, 0)

      @plsc.parallel_loop(0, 1024, 1, unroll=4)
      def _histb(i, sk=sk, shift=shift):
        k = sk[pl.ds(i * 16, 16)]
        d = lax.shift_right_logical(k, shift) & 2047
        cnt, last = plsc.scan_count(d)
        plsc.addupdate_scatter(hist_v, [d], cnt - bias + 1, mask=last)

      def cumb(i, tot):
        v = hist_v[pl.ds(i * 16, 16)]
        cin = plsc.cumsum(v)
        cnt_v[pl.ds(i * 16, 16)] = cin - v + tot
        return tot + jnp.sum(v)

      lax.fori_loop(0, 128, cumb, 0)

      def permb(i, _, sk=sk, sv=sv, dk=dk, dv=dv, shift=shift):
        k = sk[pl.ds(i * 16, 16)]
        v = sv[pl.ds(i * 16, 16)]
        d = lax.shift_right_logical(k, shift) & 2047
        cnt, last = plsc.scan_count(d)
        occ = cnt - bias
        basev = plsc.load_gather(cnt_v, [d])
        pos = basev + occ
        plsc.store_scatter(dk, [pos], k)
        plsc.store_scatter(dv, [pos], v)
        plsc.addupdate_scatter(cnt_v, [d], occ + 1, mask=last)
        return 0

      lax.fori_loop(0, 1024, permb, 0)

    pltpu.sync_copy(keys_a.at[pl.ds(0, KEEP)], ssd_out)

    def fitb(i, _):
      kk = keys_a[pl.ds(i * 16, 16)]
      fit_v[pl.ds(i * 16, 16)] = 1.0 / kk.astype(jnp.float32)
      return 0

    lax.fori_loop(0, KEEP // 16, fitb, 0)
    pltpu.sync_copy(fit_v, fit_out)
    pltpu.sync_copy(vals_a.at[pl.ds(0, KEEP)], ord_sh)

  plsc.subcore_barrier()

  pltpu.sync_copy(ord_sh.at[pl.ds(wid * 256, 256)], idx_v)
  pltpu.async_copy(gene_hbm.at[idx_v], grows_v, sem).wait()
  pltpu.sync_copy(grows_v, kept_out.at[pl.ds(wid * 256, 256)])


def _run_sort(gene_f, target_f):
  mesh = plsc.VectorSubcoreMesh(
      core_axis_name="c", subcore_axis_name="s", num_cores=1)
  return pl.kernel(
      _sort_body,
      out_type=(
          jax.ShapeDtypeStruct((KEEP, GL), jnp.float32),
          jax.ShapeDtypeStruct((KEEP,), jnp.int32),
          jax.ShapeDtypeStruct((KEEP,), jnp.float32),
      ),
      mesh=mesh,
      compiler_params=pltpu.CompilerParams(needs_layout_passes=False, use_tc_tiling_on_sc=False),
      scratch_types=[
          pltpu.VMEM_SHARED((POP,), jnp.int32),
          pltpu.VMEM_SHARED((KEEP,), jnp.int32),
          pltpu.VMEM((256, GL), jnp.float32),
          pltpu.VMEM((GL,), jnp.float32),
          pltpu.VMEM((1024,), jnp.int32),
          pltpu.VMEM((POP,), jnp.int32),
          pltpu.VMEM((POP,), jnp.int32),
          pltpu.VMEM((POP,), jnp.int32),
          pltpu.VMEM((POP,), jnp.int32),
          pltpu.VMEM((2048,), jnp.int32),
          pltpu.VMEM((2048,), jnp.int32),
          pltpu.VMEM((KEEP,), jnp.float32),
          pltpu.VMEM((256,), jnp.int32),
          pltpu.SemaphoreType.DMA,
      ],
  )(gene_f, target_f)


_ROWS_PER_W = CHILDREN // 32
_BLK = 64


def _tourn_body(noise_hbm, ssd_hbm, kept_hbm, cmask_hbm, child_out,
                pool_sh, stage_v, ssd_v, row_v2, us_v, surv_a, surv_b, set_v,
                hist_v, cum_v, ctb_v, cm_v, out_v, pid_v, pgat_v, sem, sem2):
  c = lax.axis_index("c")
  s = lax.axis_index("s")
  w = c * 16 + s
  bias = _scan_bias()

  pltpu.sync_copy(kept_hbm.at[pl.ds(s * 256, 256)], stage_v)
  pltpu.sync_copy(stage_v, pool_sh.at[pl.ds(s * 256, 256)])
  pltpu.sync_copy(ssd_hbm, ssd_v)
  plsc.subcore_barrier()

  row_base = w * _ROWS_PER_W
  pltpu.async_copy(
      noise_hbm.at[pl.ds(row_base, 1)], row_v2.at[0], sem2.at[0])

  def clear_hist():
    def clr(i, _):
      hist_v[pl.ds(i * 16, 16)] = jnp.zeros((16,), jnp.int32)
      return 0
    lax.fori_loop(0, 32, clr, 0)

  def _emit(ref, jv, em, off_v):
    cnt = plsc.cumsum(em.astype(jnp.int32))
    idx = jnp.where(em, off_v + cnt - 1, 0)
    plsc.store_scatter(ref, [idx], jv, mask=em)
    return off_v + plsc.all_reduce_population_count(em)

  def cumsum_find_l0(kneed):
    totv = jnp.zeros((16,), jnp.int32)
    base = _iota16() * 16
    for l in range(16):
      totv = totv + plsc.load_gather(hist_v, [base + l])
      totv = totv + plsc.load_gather(hist_v, [base + l + 256])
    ctot = plsc.cumsum(totv)
    ctb_v[pl.ds(0, 16)] = ctot - totv

    @plsc.parallel_loop(0, 16, 1, unroll=4)
    def _c2(ci):
      v = hist_v[pl.ds(ci * 16, 16)] + hist_v[pl.ds(ci * 16 + 256, 16)]
      cv = plsc.cumsum(v)
      basec = plsc.load_gather(ctb_v, [_splat(ci)])
      cum_v[pl.ds(ci * 16, 16)] = cv + basec

    cand = None
    for ci in range(16):
      v = cum_v[pl.ds(ci * 16, 16)]
      dd = _iota16() + ci * 16
      cc = jnp.where(v >= kneed, dd, 256)
      cand = cc if cand is None else jnp.minimum(cand, cc)
    b = jnp.min(cand)
    hb = jnp.min(plsc.load_gather(hist_v, [_splat(b)])
                 + plsc.load_gather(hist_v, [_splat(b) + 256]))
    cb = jnp.min(plsc.load_gather(cum_v, [_splat(b)]))
    return b, cb - hb

  def block_body(blk, _):
    row0 = w * _ROWS_PER_W + blk * _BLK
    pltpu.sync_copy(cmask_hbm.at[pl.ds(row0, _BLK)], cm_v)

    def row_body(r, _):
      row = row0 + r
      par = r & 1
      pltpu.make_async_copy(
          noise_hbm.at[pl.ds(row, 1)], row_v2.at[par], sem2.at[par]).wait()
      rcnt = blk * _BLK + r

      @pl.when(rcnt < _ROWS_PER_W - 1)
      def _():
        pltpu.async_copy(
            noise_hbm.at[pl.ds(row + 1, 1)], row_v2.at[1 - par],
            sem2.at[1 - par])

      clear_hist()

      @plsc.parallel_loop(0, 256, 1, unroll=4)
      def _h0(i):
        x = row_v2[par, 0, pl.ds(i * 16, 16)]
        us = _to_us(x)
        us_v[pl.ds(i * 16, 16)] = us
        d = lax.shift_right_logical(us, 24) + ((i & 1) << 8)
        cnt, last = plsc.scan_count(d)
        plsc.addupdate_scatter(hist_v, [d], cnt - bias + 1, mask=last)

      b0, below0 = cumsum_find_l0(CONT)
      kk = CONT - below0

      zero16 = jnp.zeros((16,), jnp.int32)

      @plsc.parallel_loop(0, 256, 1, unroll=4, carry=(zero16, zero16))
      def c0_carry(i, carry):
        off_set, off_srv = carry
        us = us_v[pl.ds(i * 16, 16)]
        d = lax.shift_right_logical(us, 24)
        j = _iota16() + i * 16
        em = d < b0
        sm = d == b0
        off_set = _emit(set_v, j, em, off_set)
        off_srv = _emit(surv_a, j, sm, off_srv)
        return (off_set, off_srv)

      off_set, n_src_v = c0_carry
      n_src = jnp.min(n_src_v)

      state = (off_set, n_src, kk)
      lvl_bufs = [(surv_a, surv_b), (surv_b, surv_a), (surv_a, surv_b),
                  (surv_b, surv_a)]
      for (src, dst), shift in zip(lvl_bufs, (18, 12, 6, 0)):
        off_set, n_src, kk = state
        for ci in range(4):
          hist_v[pl.ds(ci * 16, 16)] = jnp.zeros((16,), jnp.int32)
        nv = (n_src + 15) // 16

        def hb(i, _, src=src, shift=shift, n_src=n_src):
          jv = src[pl.ds(i * 16, 16)]
          valid = (_iota16() + i * 16) < n_src
          jsafe = jnp.where(valid, jv, 0)
          us = plsc.load_gather(us_v, [jsafe])
          d = lax.shift_right_logical(us, shift) & 63
          cnt, last = plsc.scan_count(d, mask=valid)
          plsc.addupdate_scatter(hist_v, [d], cnt - bias + 1,
                                 mask=last & valid)
          return 0

        lax.fori_loop(0, nv, hb, 0)
        tot = 0
        for ci in range(4):
          v = hist_v[pl.ds(ci * 16, 16)]
          cum_v[pl.ds(ci * 16, 16)] = plsc.cumsum(v) + tot
          tot = tot + jnp.sum(v)
        b = 64
        for ci in range(4):
          v = cum_v[pl.ds(ci * 16, 16)]
          dd = _iota16() + ci * 16
          b = jnp.minimum(b, jnp.min(jnp.where(v >= kk, dd, 64)))
        hb_ = jnp.min(plsc.load_gather(hist_v, [_splat(b)]))
        cb_ = jnp.min(plsc.load_gather(cum_v, [_splat(b)]))
        below = cb_ - hb_

        def cl(i, carry, src=src, dst=dst, shift=shift, n_src=n_src, b=b):
          off_set, off_srv = carry
          jv = src[pl.ds(i * 16, 16)]
          valid = (_iota16() + i * 16) < n_src
          jsafe = jnp.where(valid, jv, 0)
          us = plsc.load_gather(us_v, [jsafe])
          d = lax.shift_right_logical(us, shift) & 63
          em = valid & (d < b)
          sm = valid & (d == b)
          off_set = _emit(set_v, jv, em, off_set)
          off_srv = _emit(dst, jv, sm, off_srv)
          return (off_set, off_srv)

        off_set, n_dst_v = lax.fori_loop(
            0, nv, cl, (off_set, jnp.zeros((16,), jnp.int32)))
        state = (off_set, jnp.min(n_dst_v), kk - below)

      off_set, n_src, kk = state
      nv = (kk + 15) // 16

      def fin(i, off_set, kk=kk):
        jv = surv_a[pl.ds(i * 16, 16)]
        valid = (_iota16() + i * 16) < kk
        return _emit(set_v, jv, valid, off_set)

      lax.fori_loop(0, nv, fin, off_set)

      jvs, sds, sks, valids = [], [], [], []
      for t in range(3):
        jv = set_v[pl.ds(t * 16, 16)]
        m = (_iota16() + t * 16) < CONT
        jsafe = jnp.where(m, jv, 0)
        sd = plsc.load_gather(ssd_v, [jsafe])
        sk = plsc.load_gather(us_v, [jsafe]) ^ _I32_MIN
        jvs.append(jsafe)
        sds.append(jnp.where(m, sd, _I32_MAX))
        sks.append(jnp.where(m, sk, _I32_MAX))
        valids.append(m)

      def pick(valids):
        e = None
        for t in range(3):
          v = jnp.where(valids[t], sds[t], _I32_MAX)
          e = v if e is None else jnp.minimum(e, v)
        sd_min = jnp.min(e)
        m2 = [valids[t] & (sds[t] == sd_min) for t in range(3)]
        e = None
        for t in range(3):
          v = jnp.where(m2[t], sks[t], _I32_MAX)
          e = v if e is None else jnp.minimum(e, v)
        sk_min = jnp.min(e)
        m3 = [m2[t] & (sks[t] == sk_min) for t in range(3)]
        e = None
        for t in range(3):
          v = jnp.where(m3[t], jvs[t], 4096)
          e = v if e is None else jnp.minimum(e, v)
        return jnp.min(e)

      j1 = pick(valids)
      valids2 = [valids[t] & (jvs[t] != j1) for t in range(3)]
      j2 = pick(valids2)

      it = _iota16()
      gvec = jnp.where(it == 0, j1, j2)
      plsc.store_scatter(pid_v, [_splat(2 * r) + it], gvec, mask=it < 2)
      return 0

    lax.fori_loop(0, _BLK, row_body, 0)

    pltpu.async_copy(pool_sh.at[pid_v], pgat_v, sem).wait()

    def xover(r, _):
      for q in range(4):
        p1 = pgat_v[2 * r, pl.ds(q * 16, 16)]
        p2 = pgat_v[2 * r + 1, pl.ds(q * 16, 16)]
        cmq = cm_v[r, pl.ds(q * 16, 16)]
        out_v[r, pl.ds(q * 16, 16)] = jnp.where(cmq != 0, p1, p2)
      return 0

    lax.fori_loop(0, _BLK, xover, 0)
    pltpu.sync_copy(out_v, child_out.at[pl.ds(row0, _BLK)])
    return 0

  lax.fori_loop(0, _ROWS_PER_W // _BLK, block_body, 0)


def _run_tournament(noise, ssd_sorted, kept_pool, cmask_i):
  mesh = plsc.VectorSubcoreMesh(
      core_axis_name="c", subcore_axis_name="s", num_cores=2)
  return pl.kernel(
      _tourn_body,
      out_type=jax.ShapeDtypeStruct((CHILDREN, GL), jnp.float32),
      mesh=mesh,
      compiler_params=pltpu.CompilerParams(needs_layout_passes=False),
      scratch_types=[
          pltpu.VMEM_SHARED((KEEP, 128), jnp.float32),
          pltpu.VMEM((256, 128), jnp.float32),
          pltpu.VMEM((KEEP,), jnp.int32),
          pltpu.VMEM((2, 1, KEEP), jnp.float32),
          pltpu.VMEM((KEEP,), jnp.int32),
          pltpu.VMEM((KEEP + 32,), jnp.int32),
          pltpu.VMEM((KEEP + 32,), jnp.int32),
          pltpu.VMEM((64,), jnp.int32),
          pltpu.VMEM((512,), jnp.int32),
          pltpu.VMEM((256,), jnp.int32),
          pltpu.VMEM((16,), jnp.int32),
          pltpu.VMEM((_BLK, GL), jnp.int32),
          pltpu.VMEM((_BLK, GL), jnp.float32),
          pltpu.VMEM((2 * _BLK,), jnp.int32),
          pltpu.VMEM((2 * _BLK, 128), jnp.float32),
          pltpu.SemaphoreType.DMA,
          pltpu.SemaphoreType.DMA((2,)),
      ],
  )(noise, ssd_sorted, kept_pool, cmask_i)


_MROWS = 256


def _mut_body(pool_ref, mn_ref, mr_ref, out_ref):
  pool = pool_ref[...]
  mn = mn_ref[...]
  mr = mr_ref[...]
  b = lax.bitcast_convert_type(mn, jnp.int32)
  key = b ^ (lax.shift_right_arithmetic(b, 31) & 0x7FFFFFFF)
  lane = lax.broadcasted_iota(jnp.int32, (_MROWS, GL), 1)
  mask = jnp.zeros((_MROWS, GL), jnp.bool_)
  for k in range(4):
    nk = key[:, k:k + 1]
    lt = (key < nk) | ((key == nk) & (lane < k))
    rank = jnp.sum(lt.astype(jnp.int32), axis=1, keepdims=True)
    mask = mask | (lane == rank)
  noise = jnp.where(mr < 0.5, jnp.float32(1.0), jnp.float32(-1.0))
  out = jnp.where(mask, pool + noise, pool)
  out_ref[...] = jnp.clip(out, 0.0, 255.0)


def _run_mutate(pool_unmut, mutate_noise, mut_rand):
  spec = pl.BlockSpec((_MROWS, GL), lambda i: (i, 0))
  return pl.pallas_call(
      _mut_body,
      grid=(POP // _MROWS,),
      in_specs=[spec, spec, spec],
      out_specs=spec,
      out_shape=jax.ShapeDtypeStruct((POP, GL), jnp.float32),
  )(pool_unmut, mutate_noise, mut_rand)


def kernel(gene_pool, target_gene, crossover_mask, contender_noise,
           mutate_noise, mut_rand):
  gene_f = gene_pool.astype(jnp.float32)
  target_f = target_gene.astype(jnp.float32)
  cmask_i = crossover_mask.astype(jnp.int32)

  kept_pool, ssd_sorted, fitnesses = _run_sort(gene_f, target_f)
  kept128 = jnp.pad(kept_pool, ((0, 0), (0, 128 - GL)))
  children = _run_tournament(contender_noise, ssd_sorted, kept128, cmask_i)
  pool_unmut = jnp.concatenate([kept_pool, children], axis=0)
  pool = _run_mutate(pool_unmut, mutate_noise, mut_rand)
  return pool, fitnesses


# speedup vs baseline: 1.0430x; 1.0430x over previous
import functools

import jax
import jax.numpy as jnp
from jax import lax
from jax.experimental import pallas as pl
from jax.experimental.pallas import tpu as pltpu
from jax.experimental.pallas import tpu_sc as plsc

POP = 16384
GL = 64
KEEP = 4096
CONT = 40
CHILDREN = POP - KEEP

_I32_MIN = -2147483648
_I32_MAX = 2147483647


def _iota16():
  return lax.iota(jnp.int32, 16)


def _splat(x):
  return jnp.zeros((16,), jnp.int32) + x


def _to_us(x_f32):
  b = plsc.bitcast(x_f32, jnp.int32)
  m = lax.shift_right_arithmetic(b, 31)
  return b ^ (m | _I32_MIN)


def _scan_bias():
  zc, _ = plsc.scan_count(jnp.zeros((16,), jnp.int32))
  return jnp.max(zc) - 15


def _sort_body(gene_hbm, tgt_hbm, kept_out, ssd_out, fit_out,
               ssd_sh, ord_sh, grows_v, tgt_v, ssd_tile_v,
               keys_a, vals_a, keys_b, vals_b,
               hist_v, cnt_v, fit_v, idx_v, sem):
  wid = lax.axis_index("s")
  bias = _scan_bias()

  pltpu.sync_copy(tgt_hbm, tgt_v)
  base_row = wid * 1024
  for chunk in range(4):
    pltpu.sync_copy(gene_hbm.at[pl.ds(base_row + chunk * 256, 256)], grows_v)

    def grp_body(g16, _, chunk=chunk):
      row_ids = _iota16() + g16 * 16

      def gene_body(g, acc):
        col = plsc.load_gather(grows_v, [row_ids, _splat(g)])
        t = plsc.load_gather(tgt_v, [_splat(g)])
        d = col - t
        return acc + d * d

      acc = lax.fori_loop(0, GL, gene_body, jnp.zeros((16,), jnp.float32))
      ssd_tile_v[pl.ds(chunk * 256 + g16 * 16, 16)] = acc.astype(jnp.int32)
      return 0

    lax.fori_loop(0, 16, grp_body, 0)
  pltpu.sync_copy(ssd_tile_v, ssd_sh.at[pl.ds(base_row, 1024)])
  plsc.subcore_barrier()

  @pl.when(wid == 0)
  def _():
    pltpu.sync_copy(ssd_sh, keys_a)

    @plsc.parallel_loop(0, 1024, 1, unroll=4)
    def _init_vals(i):
      vals_a[pl.ds(i * 16, 16)] = _iota16() + i * 16

    bufs = [(keys_a, vals_a, keys_b, vals_b),
            (keys_b, vals_b, keys_a, vals_a)]
    for p, (sk, sv, dk, dv) in enumerate(bufs):
      shift = p * 11

      def clr(i, _):
        hist_v[pl.ds(i * 16, 16)] = jnp.zeros((16,), jnp.int32)
        return 0

      lax.fori_loop(0, 128, clr, 0)

      @plsc.parallel_loop(0, 1024, 1, unroll=4)
      def _histb(i, sk=sk, shift=shift):
        k = sk[pl.ds(i * 16, 16)]
        d = lax.shift_right_logical(k, shift) & 2047
        cnt, last = plsc.scan_count(d)
        plsc.addupdate_scatter(hist_v, [d], cnt - bias + 1, mask=last)

      def cumb(i, tot):
        v = hist_v[pl.ds(i * 16, 16)]
        cin = plsc.cumsum(v)
        cnt_v[pl.ds(i * 16, 16)] = cin - v + tot
        return tot + jnp.sum(v)

      lax.fori_loop(0, 128, cumb, 0)

      def permb(i, _, sk=sk, sv=sv, dk=dk, dv=dv, shift=shift):
        k = sk[pl.ds(i * 16, 16)]
        v = sv[pl.ds(i * 16, 16)]
        d = lax.shift_right_logical(k, shift) & 2047
        cnt, last = plsc.scan_count(d)
        occ = cnt - bias
        basev = plsc.load_gather(cnt_v, [d])
        pos = basev + occ
        plsc.store_scatter(dk, [pos], k)
        plsc.store_scatter(dv, [pos], v)
        plsc.addupdate_scatter(cnt_v, [d], occ + 1, mask=last)
        return 0

      lax.fori_loop(0, 1024, permb, 0)

    pltpu.sync_copy(keys_a.at[pl.ds(0, KEEP)], ssd_out)

    def fitb(i, _):
      kk = keys_a[pl.ds(i * 16, 16)]
      fit_v[pl.ds(i * 16, 16)] = 1.0 / kk.astype(jnp.float32)
      return 0

    lax.fori_loop(0, KEEP // 16, fitb, 0)
    pltpu.sync_copy(fit_v, fit_out)
    pltpu.sync_copy(vals_a.at[pl.ds(0, KEEP)], ord_sh)

  plsc.subcore_barrier()

  pltpu.sync_copy(ord_sh.at[pl.ds(wid * 256, 256)], idx_v)
  pltpu.async_copy(gene_hbm.at[idx_v], grows_v, sem).wait()
  pltpu.sync_copy(grows_v, kept_out.at[pl.ds(wid * 256, 256)])


def _run_sort(gene_f, target_f):
  mesh = plsc.VectorSubcoreMesh(
      core_axis_name="c", subcore_axis_name="s", num_cores=1)
  return pl.kernel(
      _sort_body,
      out_type=(
          jax.ShapeDtypeStruct((KEEP, GL), jnp.float32),
          jax.ShapeDtypeStruct((KEEP,), jnp.int32),
          jax.ShapeDtypeStruct((KEEP,), jnp.float32),
      ),
      mesh=mesh,
      compiler_params=pltpu.CompilerParams(needs_layout_passes=False, use_tc_tiling_on_sc=False),
      scratch_types=[
          pltpu.VMEM_SHARED((POP,), jnp.int32),
          pltpu.VMEM_SHARED((KEEP,), jnp.int32),
          pltpu.VMEM((256, GL), jnp.float32),
          pltpu.VMEM((GL,), jnp.float32),
          pltpu.VMEM((1024,), jnp.int32),
          pltpu.VMEM((POP,), jnp.int32),
          pltpu.VMEM((POP,), jnp.int32),
          pltpu.VMEM((POP,), jnp.int32),
          pltpu.VMEM((POP,), jnp.int32),
          pltpu.VMEM((2048,), jnp.int32),
          pltpu.VMEM((2048,), jnp.int32),
          pltpu.VMEM((KEEP,), jnp.float32),
          pltpu.VMEM((256,), jnp.int32),
          pltpu.SemaphoreType.DMA,
      ],
  )(gene_f, target_f)


_ROWS_PER_W = CHILDREN // 32
_BLK = 64


def _tourn_body(noise_hbm, ssd_hbm, kept_hbm, cmask_hbm, child_out,
                pool_sh, stage_v, ssd_v, row_v2, us_v, surv_a, surv_b, set_v,
                hist_v, cum_v, ctb_v, cm_v, out_v, pid_v, pgat_v, sem, sem2):
  c = lax.axis_index("c")
  s = lax.axis_index("s")
  w = c * 16 + s
  bias = _scan_bias()

  pltpu.sync_copy(kept_hbm.at[pl.ds(s * 256, 256)], stage_v)
  pltpu.sync_copy(stage_v, pool_sh.at[pl.ds(s * 256, 256)])
  pltpu.sync_copy(ssd_hbm, ssd_v)
  plsc.subcore_barrier()

  row_base = w * _ROWS_PER_W
  pltpu.async_copy(
      noise_hbm.at[pl.ds(row_base, 1)], row_v2.at[0], sem2.at[0])

  def clear_hist():
    def clr(i, _):
      hist_v[pl.ds(i * 16, 16)] = jnp.zeros((16,), jnp.int32)
      return 0
    lax.fori_loop(0, 16, clr, 0)

  def _emit(ref, jv, em, off_v):
    cnt = plsc.cumsum(em.astype(jnp.int32))
    idx = jnp.where(em, off_v + cnt - 1, 0)
    plsc.store_scatter(ref, [idx], jv, mask=em)
    return off_v + plsc.all_reduce_population_count(em)

  def cumsum_find_l0(kneed):
    totv = jnp.zeros((16,), jnp.int32)
    base = _iota16() * 16
    for l in range(16):
      totv = totv + plsc.load_gather(hist_v, [base + l])
    ctot = plsc.cumsum(totv)
    ctb_v[pl.ds(0, 16)] = ctot - totv

    @plsc.parallel_loop(0, 16, 1, unroll=4)
    def _c2(ci):
      v = hist_v[pl.ds(ci * 16, 16)]
      cv = plsc.cumsum(v)
      basec = plsc.load_gather(ctb_v, [_splat(ci)])
      cum_v[pl.ds(ci * 16, 16)] = cv + basec

    cand = None
    for ci in range(16):
      v = cum_v[pl.ds(ci * 16, 16)]
      dd = _iota16() + ci * 16
      cc = jnp.where(v >= kneed, dd, 256)
      cand = cc if cand is None else jnp.minimum(cand, cc)
    b = jnp.min(cand)
    hb = jnp.min(plsc.load_gather(hist_v, [_splat(b)]))
    cb = jnp.min(plsc.load_gather(cum_v, [_splat(b)]))
    return b, cb - hb

  def block_body(blk, _):
    row0 = w * _ROWS_PER_W + blk * _BLK
    pltpu.sync_copy(cmask_hbm.at[pl.ds(row0, _BLK)], cm_v)

    def row_body(r, _):
      row = row0 + r
      par = r & 1
      pltpu.make_async_copy(
          noise_hbm.at[pl.ds(row, 1)], row_v2.at[par], sem2.at[par]).wait()
      rcnt = blk * _BLK + r

      @pl.when(rcnt < _ROWS_PER_W - 1)
      def _():
        pltpu.async_copy(
            noise_hbm.at[pl.ds(row + 1, 1)], row_v2.at[1 - par],
            sem2.at[1 - par])

      clear_hist()

      @plsc.parallel_loop(0, 256, 1, unroll=4)
      def _h0(i):
        x = row_v2[par, 0, pl.ds(i * 16, 16)]
        us = _to_us(x)
        us_v[pl.ds(i * 16, 16)] = us
        d = lax.shift_right_logical(us, 24)
        cnt, last = plsc.scan_count(d)
        plsc.addupdate_scatter(hist_v, [d], cnt - bias + 1, mask=last)

      b0, below0 = cumsum_find_l0(CONT)
      kk = CONT - below0

      zero16 = jnp.zeros((16,), jnp.int32)

      @plsc.parallel_loop(0, 256, 1, unroll=4, carry=(zero16, zero16))
      def c0_carry(i, carry):
        off_set, off_srv = carry
        us = us_v[pl.ds(i * 16, 16)]
        d = lax.shift_right_logical(us, 24)
        j = _iota16() + i * 16
        em = d < b0
        sm = d == b0
        off_set = _emit(set_v, j, em, off_set)
        off_srv = _emit(surv_a, j, sm, off_srv)
        return (off_set, off_srv)

      off_set, n_src_v = c0_carry
      n_src = jnp.min(n_src_v)

      state = (off_set, n_src, kk)
      lvl_bufs = [(surv_a, surv_b), (surv_b, surv_a), (surv_a, surv_b),
                  (surv_b, surv_a)]
      for (src, dst), shift in zip(lvl_bufs, (18, 12, 6, 0)):
        off_set, n_src, kk = state
        for ci in range(4):
          hist_v[pl.ds(ci * 16, 16)] = jnp.zeros((16,), jnp.int32)
        nv = (n_src + 15) // 16

        def hb(i, _, src=src, shift=shift, n_src=n_src):
          jv = src[pl.ds(i * 16, 16)]
          valid = (_iota16() + i * 16) < n_src
          jsafe = jnp.where(valid, jv, 0)
          us = plsc.load_gather(us_v, [jsafe])
          d = lax.shift_right_logical(us, shift) & 63
          cnt, last = plsc.scan_count(d, mask=valid)
          plsc.addupdate_scatter(hist_v, [d], cnt - bias + 1,
                                 mask=last & valid)
          return 0

        lax.fori_loop(0, nv, hb, 0)
        tot = 0
        for ci in range(4):
          v = hist_v[pl.ds(ci * 16, 16)]
          cum_v[pl.ds(ci * 16, 16)] = plsc.cumsum(v) + tot
          tot = tot + jnp.sum(v)
        b = 64
        for ci in range(4):
          v = cum_v[pl.ds(ci * 16, 16)]
          dd = _iota16() + ci * 16
          b = jnp.minimum(b, jnp.min(jnp.where(v >= kk, dd, 64)))
        hb_ = jnp.min(plsc.load_gather(hist_v, [_splat(b)]))
        cb_ = jnp.min(plsc.load_gather(cum_v, [_splat(b)]))
        below = cb_ - hb_

        def cl(i, carry, src=src, dst=dst, shift=shift, n_src=n_src, b=b):
          off_set, off_srv = carry
          jv = src[pl.ds(i * 16, 16)]
          valid = (_iota16() + i * 16) < n_src
          jsafe = jnp.where(valid, jv, 0)
          us = plsc.load_gather(us_v, [jsafe])
          d = lax.shift_right_logical(us, shift) & 63
          em = valid & (d < b)
          sm = valid & (d == b)
          off_set = _emit(set_v, jv, em, off_set)
          off_srv = _emit(dst, jv, sm, off_srv)
          return (off_set, off_srv)

        off_set, n_dst_v = lax.fori_loop(
            0, nv, cl, (off_set, jnp.zeros((16,), jnp.int32)))
        state = (off_set, jnp.min(n_dst_v), kk - below)

      off_set, n_src, kk = state
      nv = (kk + 15) // 16

      def fin(i, off_set, kk=kk):
        jv = surv_a[pl.ds(i * 16, 16)]
        valid = (_iota16() + i * 16) < kk
        return _emit(set_v, jv, valid, off_set)

      lax.fori_loop(0, nv, fin, off_set)

      jvs, sds, sks, valids = [], [], [], []
      for t in range(3):
        jv = set_v[pl.ds(t * 16, 16)]
        m = (_iota16() + t * 16) < CONT
        jsafe = jnp.where(m, jv, 0)
        sd = plsc.load_gather(ssd_v, [jsafe])
        sk = plsc.load_gather(us_v, [jsafe]) ^ _I32_MIN
        jvs.append(jsafe)
        sds.append(jnp.where(m, sd, _I32_MAX))
        sks.append(jnp.where(m, sk, _I32_MAX))
        valids.append(m)

      def pick(valids):
        e = None
        for t in range(3):
          v = jnp.where(valids[t], sds[t], _I32_MAX)
          e = v if e is None else jnp.minimum(e, v)
        sd_min = jnp.min(e)
        m2 = [valids[t] & (sds[t] == sd_min) for t in range(3)]
        e = None
        for t in range(3):
          v = jnp.where(m2[t], sks[t], _I32_MAX)
          e = v if e is None else jnp.minimum(e, v)
        sk_min = jnp.min(e)
        m3 = [m2[t] & (sks[t] == sk_min) for t in range(3)]
        e = None
        for t in range(3):
          v = jnp.where(m3[t], jvs[t], 4096)
          e = v if e is None else jnp.minimum(e, v)
        return jnp.min(e)

      j1 = pick(valids)
      valids2 = [valids[t] & (jvs[t] != j1) for t in range(3)]
      j2 = pick(valids2)

      it = _iota16()
      gvec = jnp.where(it == 0, j1, j2)
      plsc.store_scatter(pid_v, [_splat(2 * r) + it], gvec, mask=it < 2)
      return 0

    lax.fori_loop(0, _BLK, row_body, 0)

    pltpu.async_copy(pool_sh.at[pid_v], pgat_v, sem).wait()

    def xover(r, _):
      for q in range(4):
        p1 = pgat_v[2 * r, pl.ds(q * 16, 16)]
        p2 = pgat_v[2 * r + 1, pl.ds(q * 16, 16)]
        cmq = cm_v[r, pl.ds(q * 16, 16)]
        out_v[r, pl.ds(q * 16, 16)] = jnp.where(cmq != 0, p1, p2)
      return 0

    lax.fori_loop(0, _BLK, xover, 0)
    pltpu.sync_copy(out_v, child_out.at[pl.ds(row0, _BLK)])
    return 0

  lax.fori_loop(0, _ROWS_PER_W // _BLK, block_body, 0)


def _run_tournament(noise, ssd_sorted, kept_pool, cmask_i):
  mesh = plsc.VectorSubcoreMesh(
      core_axis_name="c", subcore_axis_name="s", num_cores=2)
  return pl.kernel(
      _tourn_body,
      out_type=jax.ShapeDtypeStruct((CHILDREN, GL), jnp.float32),
      mesh=mesh,
      compiler_params=pltpu.CompilerParams(needs_layout_passes=False),
      scratch_types=[
          pltpu.VMEM_SHARED((KEEP, 128), jnp.float32),
          pltpu.VMEM((256, 128), jnp.float32),
          pltpu.VMEM((KEEP,), jnp.int32),
          pltpu.VMEM((2, 1, KEEP), jnp.float32),
          pltpu.VMEM((KEEP,), jnp.int32),
          pltpu.VMEM((KEEP + 32,), jnp.int32),
          pltpu.VMEM((KEEP + 32,), jnp.int32),
          pltpu.VMEM((64,), jnp.int32),
          pltpu.VMEM((256,), jnp.int32),
          pltpu.VMEM((256,), jnp.int32),
          pltpu.VMEM((16,), jnp.int32),
          pltpu.VMEM((_BLK, GL), jnp.int32),
          pltpu.VMEM((_BLK, GL), jnp.float32),
          pltpu.VMEM((2 * _BLK,), jnp.int32),
          pltpu.VMEM((2 * _BLK, 128), jnp.float32),
          pltpu.SemaphoreType.DMA,
          pltpu.SemaphoreType.DMA((2,)),
      ],
  )(noise, ssd_sorted, kept_pool, cmask_i)


_MROWS = 256


def _mut_body(pool_ref, mn_ref, mr_ref, out_ref):
  pool = pool_ref[...]
  mn = mn_ref[...]
  mr = mr_ref[...]
  b = lax.bitcast_convert_type(mn, jnp.int32)
  key = b ^ (lax.shift_right_arithmetic(b, 31) & 0x7FFFFFFF)
  lane = lax.broadcasted_iota(jnp.int32, (_MROWS, GL), 1)
  mask = jnp.zeros((_MROWS, GL), jnp.bool_)
  for k in range(4):
    nk = key[:, k:k + 1]
    lt = (key < nk) | ((key == nk) & (lane < k))
    rank = jnp.sum(lt.astype(jnp.int32), axis=1, keepdims=True)
    mask = mask | (lane == rank)
  noise = jnp.where(mr < 0.5, jnp.float32(1.0), jnp.float32(-1.0))
  out = jnp.where(mask, pool + noise, pool)
  out_ref[...] = jnp.clip(out, 0.0, 255.0)


def _run_mutate(pool_unmut, mutate_noise, mut_rand):
  spec = pl.BlockSpec((_MROWS, GL), lambda i: (i, 0))
  return pl.pallas_call(
      _mut_body,
      grid=(POP // _MROWS,),
      in_specs=[spec, spec, spec],
      out_specs=spec,
      out_shape=jax.ShapeDtypeStruct((POP, GL), jnp.float32),
  )(pool_unmut, mutate_noise, mut_rand)


def kernel(gene_pool, target_gene, crossover_mask, contender_noise,
           mutate_noise, mut_rand):
  gene_f = gene_pool.astype(jnp.float32)
  target_f = target_gene.astype(jnp.float32)
  cmask_i = crossover_mask.astype(jnp.int32)

  kept_pool, ssd_sorted, fitnesses = _run_sort(gene_f, target_f)
  kept128 = jnp.pad(kept_pool, ((0, 0), (0, 128 - GL)))
  children = _run_tournament(contender_noise, ssd_sorted, kept128, cmask_i)
  pool_unmut = jnp.concatenate([kept_pool, children], axis=0)
  pool = _run_mutate(pool_unmut, mutate_noise, mut_rand)
  return pool, fitnesses
